# Initial kernel scaffold; baseline (speedup 1.0000x reference)
import jax
import jax.numpy as jnp
from jax.experimental import pallas as pl


def kernel(state, lengths, v0, s0, dth, amax, b):
    raise NotImplementedError("write your pallas kernel here")


# speedup vs baseline: 75.4565x; 75.4565x over previous
import functools

import numpy as np
import jax
import jax.numpy as jnp
from jax.experimental import pallas as pl
from jax.experimental.pallas import tpu as pltpu

_COS2 = float(np.cos(np.deg2rad(20.0)) ** 2)
_LANES = 128


def _idm_body(scal_ref, x_ref, y_ref, v_ref, p_ref, o_ref, vx_ref, vy_ref,
              *, n_veh: int, n_pad: int):
    X = x_ref[...]
    Y = y_ref[...]
    V = v_ref[...]
    P = p_ref[...]
    C = jnp.cos(P)
    S = jnp.sin(P)
    VX = V * C
    VY = V * S
    vx_ref[...] = VX
    vy_ref[...] = VY
    inf = jnp.float32(np.inf)
    cnd0 = jnp.full((n_pad, _LANES), inf, jnp.float32)
    z = jnp.zeros((n_pad, _LANES), jnp.float32)

    def body(a, carry):
        cnd, lvx, lvy = carry
        xa = x_ref[pl.ds(a, 1), :]
        ya = y_ref[pl.ds(a, 1), :]
        vxa = vx_ref[pl.ds(a, 1), :]
        vya = vy_ref[pl.ds(a, 1), :]
        dx = xa - X
        dy = ya - Y
        nd = dx * C + dy * S
        dr2 = dx * dx + dy * dy
        cone = nd * jnp.abs(nd) > dr2 * _COS2
        upd = cone & (nd < cnd)
        cnd = jnp.where(upd, nd, cnd)
        lvx = jnp.where(upd, vxa, lvx)
        lvy = jnp.where(upd, vya, lvy)
        return cnd, lvx, lvy

    cnd, lvx, lvy = jax.lax.fori_loop(0, n_veh, body, (cnd0, z, z))

    L = scal_ref[0]
    v0 = scal_ref[1]
    s0 = scal_ref[2]
    dth = scal_ref[3]
    amax = scal_ref[4]
    bb = scal_ref[5]
    inv2 = 0.5 * jax.lax.rsqrt(amax * bb)
    dvx = lvx - VX
    dvy = lvy - VY
    ndv = dvx * C + dvy * S
    sstar = s0 + V * dth + V * ndv * inv2
    sal = cnd - L
    t = V * (1.0 / v0)
    t2 = t * t
    af = amax * (1.0 - t2 * t2)
    r = sstar / sal
    act = af - amax * (r * r)
    o_ref[...] = jnp.where(sal == inf, af, act)


def kernel(state, lengths, v0, s0, dth, amax, b):
    B = state.shape[0]
    st = state.reshape(B, -1, 5)
    n = st.shape[1]
    n_pad = ((n + 7) // 8) * 8
    pad = ((0, n_pad - n), (0, 0))
    xt = jnp.pad(st[..., 0].T, pad)
    yt = jnp.pad(st[..., 1].T, pad)
    vt = jnp.pad(st[..., 2].T, pad)
    pt = jnp.pad(st[..., 3].T, pad)
    scal = jnp.concatenate(
        [lengths, v0, s0, dth, amax, b]).astype(jnp.float32)

    grid = (B // _LANES,)
    body = functools.partial(_idm_body, n_veh=n, n_pad=n_pad)
    vspec = pl.BlockSpec((n_pad, _LANES), lambda i: (0, i))
    out = pl.pallas_call(
        body,
        grid=grid,
        in_specs=[
            pl.BlockSpec(memory_space=pltpu.SMEM),
            vspec, vspec, vspec, vspec,
        ],
        out_specs=vspec,
        out_shape=jax.ShapeDtypeStruct((n_pad, B), jnp.float32),
        scratch_shapes=[
            pltpu.VMEM((n_pad, _LANES), jnp.float32),
            pltpu.VMEM((n_pad, _LANES), jnp.float32),
        ],
        compiler_params=pltpu.CompilerParams(
            dimension_semantics=("parallel",)),
    )(scal, xt, yt, vt, pt)
    return out[:n].T[..., None]
